# baseline (speedup 1.0000x reference)
import jax
import jax.numpy as jnp
from jax import lax
from jax.experimental import pallas as pl
from jax.experimental.pallas import tpu as pltpu
from jax.experimental.pallas import tpu_sc as plsc

_VOCAB = 30522
_DIM = 768
_SEQ = 512
_BATCH = 128
_EPS = 1e-12

_LANES = 16
_NJ = _DIM // _LANES
_NC = 2
_NS = 16
_NW = _NC * _NS

_SEQ_CHUNKS = 8
_S_PER_W = _SEQ // _SEQ_CHUNKS
_BG = _NW // _SEQ_CHUNKS
_B_PER_W = _BATCH // _BG
_ROWS = 32
_HALVES = _S_PER_W // _ROWS
_CHUNKS = _B_PER_W * _HALVES
_PAIRS = _CHUNKS // 2

_GROUPS = _ROWS // _LANES
_CSTEP = 4

_BLK = 8
_NBLK = _NJ // _BLK


def _newton_rsqrt(v):
    i = plsc.bitcast(v, jnp.int32)
    i = jnp.full((_LANES,), 0x5F3759DF, jnp.int32) - lax.shift_right_logical(
        i, jnp.full((_LANES,), 1, jnp.int32))
    y = plsc.bitcast(i, jnp.float32)
    for _ in range(3):
        y = y * (1.5 - 0.5 * v * y * y)
    return y


def _body(news_ref, table_ref, post_ref, gamma_ref, beta_ref, out_ref,
          pos_t, idx_all, buf0, buf1, xbuf, gamma_v, beta_v, mu_v, inv_v,
          gsem0, gsem1, osem0, osem1):
    wid = lax.axis_index("s") * _NC + lax.axis_index("c")
    sc_id = wid % _SEQ_CHUNKS
    bg = wid // _SEQ_CHUNKS
    s0 = sc_id * _S_PER_W
    b0 = bg * _B_PER_W

    bufs = (buf0, buf1)
    gsems = (gsem0, gsem1)
    osems = (osem0, osem1)

    pltpu.sync_copy(post_ref.at[sc_id * _HALVES], pos_t)
    pltpu.sync_copy(news_ref.at[bg * _SEQ_CHUNKS + sc_id], idx_all)
    pltpu.sync_copy(gamma_ref, gamma_v)
    pltpu.sync_copy(beta_ref, beta_v)

    def _bl_half(c):
        return c % _B_PER_W, c // _B_PER_W

    def start_gather(c, slot):
        bl, half = _bl_half(c)
        pltpu.async_copy(
            table_ref.at[idx_all.at[pl.ds(bl * _S_PER_W + half * _ROWS,
                                          _ROWS)]],
            bufs[slot], gsems[slot])

    def drain(sem, slot):
        pltpu.make_async_copy(table_ref.at[pl.ds(0, _ROWS)], bufs[slot],
                              sem).wait()

    def compute(c, slot):
        buf_v = bufs[slot]

        iota = lax.iota(jnp.int32, _LANES)
        for g in range(_GROUPS):
            row_vec = iota + (g * _LANES)
            zero = jnp.zeros((_LANES,), jnp.float32)
            accs0 = (zero,) * _CSTEP + (zero,) * _CSTEP

            @plsc.parallel_loop(0, _DIM, _CSTEP, carry=accs0)
            def col_acc(col, accs):
                acc = list(accs[:_CSTEP])
                acc2 = list(accs[_CSTEP:])
                for d in range(_CSTEP):
                    cc = col + d
                    cvec = jnp.full((_LANES,), 0, jnp.int32) + cc
                    w = plsc.load_gather(buf_v, [row_vec, cvec])
                    x = w + pos_t[pl.ds(cc * _ROWS + g * _LANES, _LANES)]
                    plsc.store_scatter(xbuf, [row_vec, cvec], x)
                    acc[d] = acc[d] + x
                    acc2[d] = acc2[d] + x * x
                return tuple(acc) + tuple(acc2)

            accs = col_acc
            s = (accs[0] + accs[1]) + (accs[2] + accs[3])
            ss = (accs[4] + accs[5]) + (accs[6] + accs[7])
            mu = s * (1.0 / _DIM)
            var = jnp.maximum(ss * (1.0 / _DIM) - mu * mu, 0.0)
            inv = _newton_rsqrt(var + _EPS)
            for i in range(_LANES):
                mu_v[g * _LANES + i] = mu[i]
                inv_v[g * _LANES + i] = inv[i]

        for blk in range(_NBLK):
            gs = [gamma_v[pl.ds((blk * _BLK + jj) * _LANES, _LANES)]
                  for jj in range(_BLK)]
            bs = [beta_v[pl.ds((blk * _BLK + jj) * _LANES, _LANES)]
                  for jj in range(_BLK)]

            @plsc.parallel_loop(0, _ROWS, unroll=2)
            def row_norm(r):
                mu = mu_v[r]
                inv = inv_v[r]
                for jj in range(_BLK):
                    sl = pl.ds((blk * _BLK + jj) * _LANES, _LANES)
                    buf_v[r, sl] = ((xbuf[r, sl] - mu) * inv) * gs[jj] \
                        + bs[jj]

    def process(c, slot):
        other = 1 - slot
        drain(gsems[slot], slot)

        @pl.when(c + 1 < _CHUNKS)
        def _():
            @pl.when(c >= 1)
            def _():
                drain(osems[other], other)
            start_gather(c + 1, other)

        compute(c, slot)
        bl, half = _bl_half(c)
        tok0 = (b0 + bl) * _SEQ + s0 + half * _ROWS
        pltpu.async_copy(bufs[slot], out_ref.at[pl.ds(tok0, _ROWS)],
                         osems[slot])

    start_gather(0, 0)

    def pair_body(k, _):
        @pl.when(2 * k == _B_PER_W)
        def _():
            pltpu.sync_copy(post_ref.at[sc_id * _HALVES + 1], pos_t)

        process(2 * k, 0)
        process(2 * k + 1, 1)
        return 0

    lax.fori_loop(0, _PAIRS, pair_body, 0)

    drain(osem0, 0)
    drain(osem1, 1)


@jax.jit
def _embed_ln(news_r, word_table, pos_t8, gamma, beta):
    mesh = plsc.VectorSubcoreMesh(core_axis_name="c", subcore_axis_name="s")
    kfn = pl.kernel(
        _body,
        mesh=mesh,
        compiler_params=pltpu.CompilerParams(needs_layout_passes=False,
                                             use_tc_tiling_on_sc=False),
        out_type=jax.ShapeDtypeStruct((_BATCH * _SEQ, _DIM), jnp.float32),
        scratch_types=[
            pltpu.VMEM((_DIM * _ROWS,), jnp.float32),
            pltpu.VMEM((_B_PER_W * _S_PER_W,), jnp.int32),
            pltpu.VMEM((_ROWS, _DIM), jnp.float32),
            pltpu.VMEM((_ROWS, _DIM), jnp.float32),
            pltpu.VMEM((_ROWS, _DIM), jnp.float32),
            pltpu.VMEM((_DIM,), jnp.float32),
            pltpu.VMEM((_DIM,), jnp.float32),
            pltpu.SMEM((_ROWS,), jnp.float32),
            pltpu.SMEM((_ROWS,), jnp.float32),
            pltpu.SemaphoreType.DMA,
            pltpu.SemaphoreType.DMA,
            pltpu.SemaphoreType.DMA,
            pltpu.SemaphoreType.DMA,
        ],
    )
    return kfn(news_r, word_table, pos_t8, gamma, beta)


def kernel(news_batch, word_table, pos_table, gamma, beta):
    news_r = (news_batch.astype(jnp.int32)
              .reshape(_BG, _B_PER_W, _SEQ_CHUNKS, _S_PER_W)
              .transpose(0, 2, 1, 3)
              .reshape(_NW, _B_PER_W * _S_PER_W))
    pos_t8 = (pos_table.reshape(_SEQ_CHUNKS * _HALVES, _ROWS, _DIM)
              .transpose(0, 2, 1).reshape(_SEQ_CHUNKS * _HALVES,
                                          _DIM * _ROWS))
    out = _embed_ln(news_r, word_table, pos_t8, gamma, beta)
    return out.reshape(_BATCH, _SEQ, _DIM)


# speedup vs baseline: 1.7790x; 1.7790x over previous
import jax
import jax.numpy as jnp
from jax import lax
from jax.experimental import pallas as pl
from jax.experimental.pallas import tpu as pltpu
from jax.experimental.pallas import tpu_sc as plsc

_VOCAB = 30522
_DIM = 768
_SEQ = 512
_BATCH = 128
_EPS = 1e-12

_LANES = 16
_NJ = _DIM // _LANES
_NC = 2
_NS = 16
_NW = _NC * _NS

_SEQ_CHUNKS = 8
_S_PER_W = _SEQ // _SEQ_CHUNKS
_BG = _NW // _SEQ_CHUNKS
_B_PER_W = _BATCH // _BG
_ROWS = 32
_HALVES = _S_PER_W // _ROWS
_CHUNKS = _B_PER_W * _HALVES
_PAIRS = _CHUNKS // 2

_BLK = 8
_NBLK = _NJ // _BLK


def _newton_rsqrt(v):
    i = plsc.bitcast(v, jnp.int32)
    i = jnp.full((_LANES,), 0x5F3759DF, jnp.int32) - lax.shift_right_logical(
        i, jnp.full((_LANES,), 1, jnp.int32))
    y = plsc.bitcast(i, jnp.float32)
    for _ in range(3):
        y = y * (1.5 - 0.5 * v * y * y)
    return y


def _rsqrt_scalar(v_s):
    v = jnp.full((_LANES,), 0.0, jnp.float32) + v_s
    return jnp.max(_newton_rsqrt(v))


def _body(news_ref, table_ref, pos_ref, gamma_ref, beta_ref, out_ref,
          pos_v, idx_all, buf0, buf1, xbuf, gamma_v, beta_v, mu_v, inv_v,
          gsem0, gsem1, osem):
    wid = lax.axis_index("s") * _NC + lax.axis_index("c")
    sc_id = wid % _SEQ_CHUNKS
    bg = wid // _SEQ_CHUNKS
    s0 = sc_id * _S_PER_W
    b0 = bg * _B_PER_W

    bufs = (buf0, buf1)
    gsems = (gsem0, gsem1)

    pltpu.sync_copy(news_ref.at[bg * _SEQ_CHUNKS + sc_id], idx_all)
    pltpu.sync_copy(pos_ref.at[pl.ds(s0, _ROWS)], pos_v)
    pltpu.sync_copy(gamma_ref, gamma_v)
    pltpu.sync_copy(beta_ref, beta_v)

    def _bl_half(c):
        return c % _B_PER_W, c // _B_PER_W

    def start_gather(c, slot):
        bl, half = _bl_half(c)
        pltpu.async_copy(
            table_ref.at[idx_all.at[pl.ds(bl * _S_PER_W + half * _ROWS,
                                          _ROWS)]],
            bufs[slot], gsems[slot])

    def drain(sem, slot):
        pltpu.make_async_copy(table_ref.at[pl.ds(0, _ROWS)], bufs[slot],
                              sem).wait()

    def process(c, slot):
        other = 1 - slot

        @pl.when(c == _B_PER_W)
        def _():
            pltpu.sync_copy(pos_ref.at[pl.ds(s0 + _ROWS, _ROWS)], pos_v)

        @pl.when(c + 1 < _CHUNKS)
        def _():
            start_gather(c + 1, other)

        drain(gsems[slot], slot)
        buf_v = bufs[slot]

        @plsc.parallel_loop(0, _ROWS)
        def row_stats(r):
            acc = [jnp.zeros((_LANES,), jnp.float32) for _ in range(4)]
            acc2 = [jnp.zeros((_LANES,), jnp.float32) for _ in range(4)]
            for j in range(_NJ):
                sl = pl.ds(j * _LANES, _LANES)
                x = buf_v[r, sl] + pos_v[r, sl]
                acc[j % 4] = acc[j % 4] + x
                acc2[j % 4] = acc2[j % 4] + x * x
            s = jnp.sum((acc[0] + acc[1]) + (acc[2] + acc[3]))
            ss = jnp.sum((acc2[0] + acc2[1]) + (acc2[2] + acc2[3]))
            mu = s * (1.0 / _DIM)
            var = jnp.maximum(ss * (1.0 / _DIM) - mu * mu, 0.0)
            mu_v[r] = mu
            inv_v[r] = _rsqrt_scalar(var + _EPS)

        @pl.when(c >= 1)
        def _():
            drain(osem, slot)

        for blk in range(_NBLK):
            gs = [gamma_v[pl.ds((blk * _BLK + jj) * _LANES, _LANES)]
                  for jj in range(_BLK)]
            bs = [beta_v[pl.ds((blk * _BLK + jj) * _LANES, _LANES)]
                  for jj in range(_BLK)]

            @plsc.parallel_loop(0, _ROWS, unroll=2)
            def row_norm(r):
                mu = mu_v[r]
                inv = inv_v[r]
                for jj in range(_BLK):
                    sl = pl.ds((blk * _BLK + jj) * _LANES, _LANES)
                    x = buf_v[r, sl] + pos_v[r, sl]
                    xbuf[r, sl] = ((x - mu) * inv) * gs[jj] + bs[jj]

        bl, half = _bl_half(c)
        tok0 = (b0 + bl) * _SEQ + s0 + half * _ROWS
        pltpu.async_copy(xbuf, out_ref.at[pl.ds(tok0, _ROWS)], osem)

    start_gather(0, 0)

    def pair_body(k, _):
        process(2 * k, 0)
        process(2 * k + 1, 1)
        return 0

    lax.fori_loop(0, _PAIRS, pair_body, 0)

    drain(osem, 0)


@jax.jit
def _embed_ln(news_r, word_table, pos_table, gamma, beta):
    mesh = plsc.VectorSubcoreMesh(core_axis_name="c", subcore_axis_name="s")
    kfn = pl.kernel(
        _body,
        mesh=mesh,
        compiler_params=pltpu.CompilerParams(needs_layout_passes=False,
                                             use_tc_tiling_on_sc=False),
        out_type=jax.ShapeDtypeStruct((_BATCH * _SEQ, _DIM), jnp.float32),
        scratch_types=[
            pltpu.VMEM((_ROWS, _DIM), jnp.float32),
            pltpu.VMEM((_B_PER_W * _S_PER_W,), jnp.int32),
            pltpu.VMEM((_ROWS, _DIM), jnp.float32),
            pltpu.VMEM((_ROWS, _DIM), jnp.float32),
            pltpu.VMEM((_ROWS, _DIM), jnp.float32),
            pltpu.VMEM((_DIM,), jnp.float32),
            pltpu.VMEM((_DIM,), jnp.float32),
            pltpu.SMEM((_ROWS,), jnp.float32),
            pltpu.SMEM((_ROWS,), jnp.float32),
            pltpu.SemaphoreType.DMA,
            pltpu.SemaphoreType.DMA,
            pltpu.SemaphoreType.DMA,
        ],
    )
    return kfn(news_r, word_table, pos_table, gamma, beta)


def kernel(news_batch, word_table, pos_table, gamma, beta):
    news_r = (news_batch.astype(jnp.int32)
              .reshape(_BG, _B_PER_W, _SEQ_CHUNKS, _S_PER_W)
              .transpose(0, 2, 1, 3)
              .reshape(_NW, _B_PER_W * _S_PER_W))
    out = _embed_ln(news_r, word_table, pos_table, gamma, beta)
    return out.reshape(_BATCH, _SEQ, _DIM)


# speedup vs baseline: 3.8426x; 2.1599x over previous
import jax
import jax.numpy as jnp
from jax import lax
from jax.experimental import pallas as pl
from jax.experimental.pallas import tpu as pltpu
from jax.experimental.pallas import tpu_sc as plsc

_VOCAB = 30522
_DIM = 768
_SEQ = 512
_BATCH = 128
_EPS = 1e-12

_LANES = 16
_NJ = _DIM // _LANES
_NC = 2
_NS = 16
_NW = _NC * _NS

_SEQ_CHUNKS = 8
_S_PER_W = _SEQ // _SEQ_CHUNKS
_BG = _NW // _SEQ_CHUNKS
_B_PER_W = _BATCH // _BG
_ROWS = 32
_HALVES = _S_PER_W // _ROWS
_CHUNKS = _B_PER_W * _HALVES
_PAIRS = _CHUNKS // 2

_BLK = 8
_NBLK = _NJ // _BLK


def _newton_rsqrt(v):
    i = plsc.bitcast(v, jnp.int32)
    i = jnp.full((_LANES,), 0x5F3759DF, jnp.int32) - lax.shift_right_logical(
        i, jnp.full((_LANES,), 1, jnp.int32))
    y = plsc.bitcast(i, jnp.float32)
    for _ in range(3):
        y = y * (1.5 - 0.5 * v * y * y)
    return y


def _rsqrt_scalar(v_s):
    v = jnp.full((_LANES,), 0.0, jnp.float32) + v_s
    return jnp.max(_newton_rsqrt(v))


def _body(news_ref, table_ref, pos_ref, gamma_ref, beta_ref, out_ref,
          pos_v, idx_all, buf0, buf1, xbuf, gamma_v, beta_v, mu_v, inv_v,
          gsem0, gsem1, osem):
    wid = lax.axis_index("s") * _NC + lax.axis_index("c")
    sc_id = wid % _SEQ_CHUNKS
    bg = wid // _SEQ_CHUNKS
    s0 = sc_id * _S_PER_W
    b0 = bg * _B_PER_W

    bufs = (buf0, buf1)
    gsems = (gsem0, gsem1)

    pltpu.sync_copy(news_ref.at[bg * _SEQ_CHUNKS + sc_id], idx_all)
    pltpu.sync_copy(pos_ref.at[pl.ds(s0, _ROWS)], pos_v)
    pltpu.sync_copy(gamma_ref, gamma_v)
    pltpu.sync_copy(beta_ref, beta_v)

    def _bl_half(c):
        return c % _B_PER_W, c // _B_PER_W

    def start_gather(c, slot):
        bl, half = _bl_half(c)
        pltpu.async_copy(
            table_ref.at[idx_all.at[pl.ds(bl * _S_PER_W + half * _ROWS,
                                          _ROWS)]],
            bufs[slot], gsems[slot])

    def drain(sem, slot):
        pltpu.make_async_copy(table_ref.at[pl.ds(0, _ROWS)], bufs[slot],
                              sem).wait()

    def process(c, slot):
        other = 1 - slot

        @pl.when(c == _B_PER_W)
        def _():
            pltpu.sync_copy(pos_ref.at[pl.ds(s0 + _ROWS, _ROWS)], pos_v)

        @pl.when(c + 1 < _CHUNKS)
        def _():
            start_gather(c + 1, other)

        drain(gsems[slot], slot)
        buf_v = bufs[slot]

        @plsc.parallel_loop(0, _ROWS)
        def row_stats(r):
            acc = [jnp.zeros((_LANES,), jnp.float32) for _ in range(4)]
            acc2 = [jnp.zeros((_LANES,), jnp.float32) for _ in range(4)]
            for j in range(_NJ):
                sl = pl.ds(j * _LANES, _LANES)
                x = buf_v[r, sl] + pos_v[r, sl]
                acc[j % 4] = acc[j % 4] + x
                acc2[j % 4] = acc2[j % 4] + x * x
            s = jnp.sum((acc[0] + acc[1]) + (acc[2] + acc[3]))
            ss = jnp.sum((acc2[0] + acc2[1]) + (acc2[2] + acc2[3]))
            mu = s * (1.0 / _DIM)
            var = jnp.maximum(ss * (1.0 / _DIM) - mu * mu, 0.0)
            mu_v[r] = mu
            inv_v[r] = _rsqrt_scalar(var + _EPS)

        @pl.when(c >= 1)
        def _():
            drain(osem, slot)

        for blk in range(_NBLK):
            gs = [gamma_v[pl.ds((blk * _BLK + jj) * _LANES, _LANES)]
                  for jj in range(_BLK)]
            bs = [beta_v[pl.ds((blk * _BLK + jj) * _LANES, _LANES)]
                  for jj in range(_BLK)]

            @plsc.parallel_loop(0, _ROWS, unroll=2)
            def row_norm(r):
                mu = mu_v[r]
                inv = inv_v[r]
                for jj in range(_BLK):
                    sl = pl.ds((blk * _BLK + jj) * _LANES, _LANES)
                    x = buf_v[r, sl] + pos_v[r, sl]
                    xbuf[r, sl] = ((x - mu) * inv) * gs[jj] + bs[jj]

        bl, half = _bl_half(c)
        tok0 = (b0 + bl) * _SEQ + s0 + half * _ROWS
        pltpu.async_copy(xbuf, out_ref.at[pl.ds(tok0, _ROWS)], osem)

    start_gather(0, 0)

    def pair_body(k, _):
        process(2 * k, 0)
        process(2 * k + 1, 1)
        return 0

    lax.fori_loop(0, _PAIRS, pair_body, 0)

    drain(osem, 0)


@jax.jit
def _embed_ln(news_r, word_table, pos_table, gamma, beta):
    mesh = plsc.VectorSubcoreMesh(core_axis_name="c", subcore_axis_name="s")
    kfn = pl.kernel(
        _body,
        mesh=mesh,
        compiler_params=pltpu.CompilerParams(needs_layout_passes=False),
        out_type=jax.ShapeDtypeStruct((_BATCH * _SEQ, _DIM), jnp.float32),
        scratch_types=[
            pltpu.VMEM((_ROWS, _DIM), jnp.float32),
            pltpu.VMEM((_B_PER_W * _S_PER_W,), jnp.int32),
            pltpu.VMEM((_ROWS, _DIM), jnp.float32),
            pltpu.VMEM((_ROWS, _DIM), jnp.float32),
            pltpu.VMEM((_ROWS, _DIM), jnp.float32),
            pltpu.VMEM((_DIM,), jnp.float32),
            pltpu.VMEM((_DIM,), jnp.float32),
            pltpu.SMEM((_ROWS,), jnp.float32),
            pltpu.SMEM((_ROWS,), jnp.float32),
            pltpu.SemaphoreType.DMA,
            pltpu.SemaphoreType.DMA,
            pltpu.SemaphoreType.DMA,
        ],
    )
    return kfn(news_r, word_table, pos_table, gamma, beta)


def kernel(news_batch, word_table, pos_table, gamma, beta):
    news_r = (news_batch.astype(jnp.int32)
              .reshape(_BG, _B_PER_W, _SEQ_CHUNKS, _S_PER_W)
              .transpose(0, 2, 1, 3)
              .reshape(_NW, _B_PER_W * _S_PER_W))
    out = _embed_ln(news_r, word_table, pos_table, gamma, beta)
    return out.reshape(_BATCH, _SEQ, _DIM)


# speedup vs baseline: 4.7790x; 1.2437x over previous
import jax
import jax.numpy as jnp
from jax import lax
from jax.experimental import pallas as pl
from jax.experimental.pallas import tpu as pltpu
from jax.experimental.pallas import tpu_sc as plsc

_VOCAB = 30522
_DIM = 768
_SEQ = 512
_BATCH = 128
_EPS = 1e-12

_LANES = 16
_NJ = _DIM // _LANES
_NC = 2
_NS = 16
_NW = _NC * _NS

_SEQ_CHUNKS = 8
_S_PER_W = _SEQ // _SEQ_CHUNKS
_BG = _NW // _SEQ_CHUNKS
_B_PER_W = _BATCH // _BG
_ROWS = 32
_HALVES = _S_PER_W // _ROWS
_CHUNKS = _B_PER_W * _HALVES
_PAIRS = _CHUNKS // 2

_BLK = 8
_NBLK = _NJ // _BLK


def _newton_rsqrt(v):
    i = plsc.bitcast(v, jnp.int32)
    i = jnp.full((_LANES,), 0x5F3759DF, jnp.int32) - lax.shift_right_logical(
        i, jnp.full((_LANES,), 1, jnp.int32))
    y = plsc.bitcast(i, jnp.float32)
    for _ in range(3):
        y = y * (1.5 - 0.5 * v * y * y)
    return y


def _rsqrt_scalar(v_s):
    i = lax.bitcast_convert_type(v_s, jnp.int32)
    i = jnp.int32(0x5F3759DF) - lax.shift_right_logical(i, 1)
    y = lax.bitcast_convert_type(i, jnp.float32)
    for _ in range(3):
        y = y * (1.5 - 0.5 * v_s * y * y)
    return y


def _body(news_ref, table_ref, pos_ref, gamma_ref, beta_ref, out_ref,
          pos_v, idx_all, buf0, buf1, xbuf, gamma_v, beta_v, mu_v, inv_v,
          gsem0, gsem1, osem):
    wid = lax.axis_index("s") * _NC + lax.axis_index("c")
    sc_id = wid % _SEQ_CHUNKS
    bg = wid // _SEQ_CHUNKS
    s0 = sc_id * _S_PER_W
    b0 = bg * _B_PER_W

    bufs = (buf0, buf1)
    gsems = (gsem0, gsem1)

    pltpu.sync_copy(news_ref.at[bg * _SEQ_CHUNKS + sc_id], idx_all)
    pltpu.sync_copy(pos_ref.at[pl.ds(s0, _ROWS)], pos_v)
    pltpu.sync_copy(gamma_ref, gamma_v)
    pltpu.sync_copy(beta_ref, beta_v)

    def _bl_half(c):
        return c % _B_PER_W, c // _B_PER_W

    def start_gather(c, slot):
        bl, half = _bl_half(c)
        pltpu.async_copy(
            table_ref.at[idx_all.at[pl.ds(bl * _S_PER_W + half * _ROWS,
                                          _ROWS)]],
            bufs[slot], gsems[slot])

    def drain(sem, slot):
        pltpu.make_async_copy(table_ref.at[pl.ds(0, _ROWS)], bufs[slot],
                              sem).wait()

    def process(c, slot):
        other = 1 - slot

        @pl.when(c == _B_PER_W)
        def _():
            pltpu.sync_copy(pos_ref.at[pl.ds(s0 + _ROWS, _ROWS)], pos_v)

        @pl.when(c + 1 < _CHUNKS)
        def _():
            start_gather(c + 1, other)

        drain(gsems[slot], slot)
        buf_v = bufs[slot]

        @plsc.parallel_loop(0, _ROWS)
        def row_stats(r):
            acc = [jnp.zeros((_LANES,), jnp.float32) for _ in range(4)]
            acc2 = [jnp.zeros((_LANES,), jnp.float32) for _ in range(4)]
            for j in range(_NJ):
                sl = pl.ds(j * _LANES, _LANES)
                x = buf_v[r, sl] + pos_v[r, sl]
                acc[j % 4] = acc[j % 4] + x
                acc2[j % 4] = acc2[j % 4] + x * x
            s = jnp.sum((acc[0] + acc[1]) + (acc[2] + acc[3]))
            ss = jnp.sum((acc2[0] + acc2[1]) + (acc2[2] + acc2[3]))
            mu = s * (1.0 / _DIM)
            var = jnp.maximum(ss * (1.0 / _DIM) - mu * mu, 0.0)
            mu_v[r] = mu
            inv_v[r] = _rsqrt_scalar(var + _EPS)

        @pl.when(c >= 1)
        def _():
            drain(osem, slot)

        for blk in range(_NBLK):
            gs = [gamma_v[pl.ds((blk * _BLK + jj) * _LANES, _LANES)]
                  for jj in range(_BLK)]
            bs = [beta_v[pl.ds((blk * _BLK + jj) * _LANES, _LANES)]
                  for jj in range(_BLK)]

            @plsc.parallel_loop(0, _ROWS, unroll=2)
            def row_norm(r):
                mu = mu_v[r]
                inv = inv_v[r]
                for jj in range(_BLK):
                    sl = pl.ds((blk * _BLK + jj) * _LANES, _LANES)
                    x = buf_v[r, sl] + pos_v[r, sl]
                    xbuf[r, sl] = ((x - mu) * inv) * gs[jj] + bs[jj]

        bl, half = _bl_half(c)
        tok0 = (b0 + bl) * _SEQ + s0 + half * _ROWS
        pltpu.async_copy(xbuf, out_ref.at[pl.ds(tok0, _ROWS)], osem)

    start_gather(0, 0)

    def pair_body(k, _):
        process(2 * k, 0)
        process(2 * k + 1, 1)
        return 0

    lax.fori_loop(0, _PAIRS, pair_body, 0)

    drain(osem, 0)


@jax.jit
def _embed_ln(news_r, word_table, pos_table, gamma, beta):
    mesh = plsc.VectorSubcoreMesh(core_axis_name="c", subcore_axis_name="s")
    kfn = pl.kernel(
        _body,
        mesh=mesh,
        compiler_params=pltpu.CompilerParams(needs_layout_passes=False),
        out_type=jax.ShapeDtypeStruct((_BATCH * _SEQ, _DIM), jnp.float32),
        scratch_types=[
            pltpu.VMEM((_ROWS, _DIM), jnp.float32),
            pltpu.VMEM((_B_PER_W * _S_PER_W,), jnp.int32),
            pltpu.VMEM((_ROWS, _DIM), jnp.float32),
            pltpu.VMEM((_ROWS, _DIM), jnp.float32),
            pltpu.VMEM((_ROWS, _DIM), jnp.float32),
            pltpu.VMEM((_DIM,), jnp.float32),
            pltpu.VMEM((_DIM,), jnp.float32),
            pltpu.SMEM((_ROWS,), jnp.float32),
            pltpu.SMEM((_ROWS,), jnp.float32),
            pltpu.SemaphoreType.DMA,
            pltpu.SemaphoreType.DMA,
            pltpu.SemaphoreType.DMA,
        ],
    )
    return kfn(news_r, word_table, pos_table, gamma, beta)


def kernel(news_batch, word_table, pos_table, gamma, beta):
    news_r = (news_batch.astype(jnp.int32)
              .reshape(_BG, _B_PER_W, _SEQ_CHUNKS, _S_PER_W)
              .transpose(0, 2, 1, 3)
              .reshape(_NW, _B_PER_W * _S_PER_W))
    out = _embed_ln(news_r, word_table, pos_table, gamma, beta)
    return out.reshape(_BATCH, _SEQ, _DIM)


# speedup vs baseline: 4.7986x; 1.0041x over previous
import jax
import jax.numpy as jnp
from jax import lax
from jax.experimental import pallas as pl
from jax.experimental.pallas import tpu as pltpu
from jax.experimental.pallas import tpu_sc as plsc

_VOCAB = 30522
_DIM = 768
_SEQ = 512
_BATCH = 128
_EPS = 1e-12

_LANES = 16
_NJ = _DIM // _LANES
_NC = 2
_NS = 16
_NW = _NC * _NS

_SEQ_CHUNKS = 8
_S_PER_W = _SEQ // _SEQ_CHUNKS
_BG = _NW // _SEQ_CHUNKS
_B_PER_W = _BATCH // _BG
_ROWS = 32
_HALVES = _S_PER_W // _ROWS
_CHUNKS = _B_PER_W * _HALVES
_PAIRS = _CHUNKS // 2

_BLK = 8
_NBLK = _NJ // _BLK


def _newton_rsqrt(v):
    i = plsc.bitcast(v, jnp.int32)
    i = jnp.full((_LANES,), 0x5F3759DF, jnp.int32) - lax.shift_right_logical(
        i, jnp.full((_LANES,), 1, jnp.int32))
    y = plsc.bitcast(i, jnp.float32)
    for _ in range(3):
        y = y * (1.5 - 0.5 * v * y * y)
    return y


def _rsqrt_scalar(v_s):
    i = lax.bitcast_convert_type(v_s, jnp.int32)
    i = jnp.int32(0x5F3759DF) - lax.shift_right_logical(i, 1)
    y = lax.bitcast_convert_type(i, jnp.float32)
    for _ in range(3):
        y = y * (1.5 - 0.5 * v_s * y * y)
    return y


def _body(news_ref, table_ref, pos_ref, gamma_ref, beta_ref, out_ref,
          pos_v, idx_all, buf0, buf1, xbuf0, xbuf1, gamma_v, beta_v,
          mu_v, inv_v, gsem0, gsem1, osem0, osem1):
    wid = lax.axis_index("s") * _NC + lax.axis_index("c")
    sc_id = wid % _SEQ_CHUNKS
    bg = wid // _SEQ_CHUNKS
    s0 = sc_id * _S_PER_W
    b0 = bg * _B_PER_W

    bufs = (buf0, buf1)
    xbufs = (xbuf0, xbuf1)
    gsems = (gsem0, gsem1)
    osems = (osem0, osem1)

    pltpu.sync_copy(news_ref.at[bg * _SEQ_CHUNKS + sc_id], idx_all)
    pltpu.sync_copy(pos_ref.at[pl.ds(s0, _ROWS)], pos_v)
    pltpu.sync_copy(gamma_ref, gamma_v)
    pltpu.sync_copy(beta_ref, beta_v)

    def _bl_half(c):
        return c % _B_PER_W, c // _B_PER_W

    def start_gather(c, slot):
        bl, half = _bl_half(c)
        pltpu.async_copy(
            table_ref.at[idx_all.at[pl.ds(bl * _S_PER_W + half * _ROWS,
                                          _ROWS)]],
            bufs[slot], gsems[slot])

    def drain(sem, slot):
        pltpu.make_async_copy(table_ref.at[pl.ds(0, _ROWS)], bufs[slot],
                              sem).wait()

    def process(c, slot):
        other = 1 - slot

        @pl.when(c == _B_PER_W)
        def _():
            pltpu.sync_copy(pos_ref.at[pl.ds(s0 + _ROWS, _ROWS)], pos_v)

        @pl.when(c + 1 < _CHUNKS)
        def _():
            start_gather(c + 1, other)

        @pl.when(c >= 2)
        def _():
            drain(osems[slot], slot)

        drain(gsems[slot], slot)
        buf_v = bufs[slot]
        xbuf_v = xbufs[slot]

        @plsc.parallel_loop(0, _ROWS)
        def row_stats(r):
            acc = [jnp.zeros((_LANES,), jnp.float32) for _ in range(4)]
            acc2 = [jnp.zeros((_LANES,), jnp.float32) for _ in range(4)]
            for j in range(_NJ):
                sl = pl.ds(j * _LANES, _LANES)
                x = buf_v[r, sl] + pos_v[r, sl]
                acc[j % 4] = acc[j % 4] + x
                acc2[j % 4] = acc2[j % 4] + x * x
            s = jnp.sum((acc[0] + acc[1]) + (acc[2] + acc[3]))
            ss = jnp.sum((acc2[0] + acc2[1]) + (acc2[2] + acc2[3]))
            mu = s * (1.0 / _DIM)
            var = jnp.maximum(ss * (1.0 / _DIM) - mu * mu, 0.0)
            mu_v[r] = mu
            inv_v[r] = _rsqrt_scalar(var + _EPS)

        for blk in range(_NBLK):
            gs = [gamma_v[pl.ds((blk * _BLK + jj) * _LANES, _LANES)]
                  for jj in range(_BLK)]
            bs = [beta_v[pl.ds((blk * _BLK + jj) * _LANES, _LANES)]
                  for jj in range(_BLK)]

            @plsc.parallel_loop(0, _ROWS, unroll=2)
            def row_norm(r):
                mu = mu_v[r]
                inv = inv_v[r]
                for jj in range(_BLK):
                    sl = pl.ds((blk * _BLK + jj) * _LANES, _LANES)
                    x = buf_v[r, sl] + pos_v[r, sl]
                    xbuf_v[r, sl] = ((x - mu) * inv) * gs[jj] + bs[jj]

        bl, half = _bl_half(c)
        tok0 = (b0 + bl) * _SEQ + s0 + half * _ROWS
        pltpu.async_copy(xbuf_v, out_ref.at[pl.ds(tok0, _ROWS)],
                         osems[slot])

    start_gather(0, 0)

    def pair_body(k, _):
        process(2 * k, 0)
        process(2 * k + 1, 1)
        return 0

    lax.fori_loop(0, _PAIRS, pair_body, 0)

    drain(osems[0], 0)
    drain(osems[1], 1)


@jax.jit
def _embed_ln(news_r, word_table, pos_table, gamma, beta):
    mesh = plsc.VectorSubcoreMesh(core_axis_name="c", subcore_axis_name="s")
    kfn = pl.kernel(
        _body,
        mesh=mesh,
        compiler_params=pltpu.CompilerParams(needs_layout_passes=False),
        out_type=jax.ShapeDtypeStruct((_BATCH * _SEQ, _DIM), jnp.float32),
        scratch_types=[
            pltpu.VMEM((_ROWS, _DIM), jnp.float32),
            pltpu.VMEM((_B_PER_W * _S_PER_W,), jnp.int32),
            pltpu.VMEM((_ROWS, _DIM), jnp.float32),
            pltpu.VMEM((_ROWS, _DIM), jnp.float32),
            pltpu.VMEM((_ROWS, _DIM), jnp.float32),
            pltpu.VMEM((_ROWS, _DIM), jnp.float32),
            pltpu.VMEM((_DIM,), jnp.float32),
            pltpu.VMEM((_DIM,), jnp.float32),
            pltpu.SMEM((_ROWS,), jnp.float32),
            pltpu.SMEM((_ROWS,), jnp.float32),
            pltpu.SemaphoreType.DMA,
            pltpu.SemaphoreType.DMA,
            pltpu.SemaphoreType.DMA,
            pltpu.SemaphoreType.DMA,
        ],
    )
    return kfn(news_r, word_table, pos_table, gamma, beta)


def kernel(news_batch, word_table, pos_table, gamma, beta):
    news_r = (news_batch.astype(jnp.int32)
              .reshape(_BG, _B_PER_W, _SEQ_CHUNKS, _S_PER_W)
              .transpose(0, 2, 1, 3)
              .reshape(_NW, _B_PER_W * _S_PER_W))
    out = _embed_ln(news_r, word_table, pos_table, gamma, beta)
    return out.reshape(_BATCH, _SEQ, _DIM)
